# baseline (speedup 1.0000x reference)
import functools

import jax
import jax.numpy as jnp
from jax import lax
from jax.experimental import pallas as pl
from jax.experimental.pallas import tpu as pltpu
from jax.experimental.pallas import tpu_sc as plsc

_B = 4096
_N_EMB = 26
_N_DENSE = 13
_VOCAB = 100000
_EMB_DIM = 32
_OUT_DIM = 128
_EMB_COLS = _N_EMB * _EMB_DIM
_PACK = 128 // _EMB_DIM
_VP = _N_EMB * _VOCAB // _PACK

_NC, _NS = 2, 16
_NW = _NC * _NS
_BPW = _B // _NW
_RB = 4
_KC = _RB * _N_EMB
_NCH = _BPW // _RB
_L = 16
_NG = -(-_KC // _L)

_sc_mesh = plsc.VectorSubcoreMesh(core_axis_name="c", subcore_axis_name="s")


@functools.partial(
    pl.kernel,
    out_type=jax.ShapeDtypeStruct((_B, _EMB_COLS), jnp.float32),
    mesh=_sc_mesh,
    scratch_types=[
        pltpu.VMEM((_NCH, _KC), jnp.int32),
        pltpu.VMEM((_NG * _L,), jnp.int32),
        pltpu.VMEM((3, _NG, _L), jnp.int32),
        pltpu.VMEM((_KC, 128), jnp.float32),
        pltpu.VMEM((_RB, _EMB_COLS), jnp.float32),
        pltpu.SemaphoreType.DMA,
    ],
    compiler_params=pltpu.CompilerParams(needs_layout_passes=False,
                                         use_tc_tiling_on_sc=False),
)
def _sc_gather(qidx_hbm, sub_hbm, ext_hbm, tab_hbm, out_hbm,
               qidx_v, sub_v, ext_v, slab_v, row_v, sem):
    wid = lax.axis_index("s") * _NC + lax.axis_index("c")
    pltpu.sync_copy(qidx_hbm.at[wid], qidx_v)
    pltpu.sync_copy(ext_hbm, ext_v)

    def chunk_body(c, carry):
        pltpu.sync_copy(sub_hbm.at[wid, c], sub_v)
        pltpu.async_copy(tab_hbm.at[qidx_v.at[c]], slab_v, sem).wait()
        for q in range(_NG):
            k_vec = ext_v[0, q, :]
            b_vec = ext_v[1, q, :]
            col0 = ext_v[2, q, :]
            sub_vec = sub_v[pl.ds(q * _L, _L)]
            for dd in range(_EMB_DIM):
                v = plsc.load_gather(slab_v, [k_vec, sub_vec + dd])
                plsc.store_scatter(row_v, [b_vec, col0 + dd], v)
        row0 = wid * _BPW + c * _RB
        pltpu.sync_copy(row_v, out_hbm.at[pl.ds(row0, _RB)])
        return carry

    lax.fori_loop(0, _NCH, chunk_body, 0)


def _mm_body(emb_ref, dense_ref, w1_ref, w2_ref, b_ref, o_ref):
    acc = jnp.dot(
        emb_ref[...], w1_ref[...],
        preferred_element_type=jnp.float32,
        precision=lax.Precision.HIGHEST,
    )
    acc = acc + jnp.dot(
        dense_ref[...], w2_ref[...],
        preferred_element_type=jnp.float32,
        precision=lax.Precision.HIGHEST,
    )
    o_ref[...] = acc + b_ref[...]


_BM = 512


def _tc_project(emb, dense, w1, w2, b2):
    grid = (_B // _BM,)
    return pl.pallas_call(
        _mm_body,
        grid=grid,
        in_specs=[
            pl.BlockSpec((_BM, _EMB_COLS), lambda i: (i, 0)),
            pl.BlockSpec((_BM, _N_DENSE), lambda i: (i, 0)),
            pl.BlockSpec((_EMB_COLS, _OUT_DIM), lambda i: (0, 0)),
            pl.BlockSpec((_N_DENSE, _OUT_DIM), lambda i: (0, 0)),
            pl.BlockSpec((1, _OUT_DIM), lambda i: (0, 0)),
        ],
        out_specs=pl.BlockSpec((_BM, _OUT_DIM), lambda i: (i, 0)),
        out_shape=jax.ShapeDtypeStruct((_B, _OUT_DIM), jnp.float32),
    )(emb, dense, w1, w2, b2)


def _ext_consts():
    ks = [min(q * _L + l, _KC - 1) for q in range(_NG) for l in range(_L)]
    k = [ks[i * _L:(i + 1) * _L] for i in range(_NG)]
    b = [[kk // _N_EMB for kk in row] for row in k]
    c = [[(kk % _N_EMB) * _EMB_DIM for kk in row] for row in k]
    return jnp.array([k, b, c], dtype=jnp.int32)


def kernel(x, tables, W, b):
    idx = x[:, :_N_EMB].astype(jnp.int32)
    flat = idx + (jnp.arange(_N_EMB, dtype=jnp.int32) * _VOCAB)[None, :]
    qidx = (flat >> 2).reshape(_NW, _NCH, _KC)
    sub32 = ((flat & 3) * _EMB_DIM).reshape(_NW, _NCH, _KC)
    sub32 = jnp.pad(sub32, ((0, 0), (0, 0), (0, _NG * _L - _KC)), mode="edge")
    tabp = tables.reshape(_VP, _PACK * _EMB_DIM)
    emb = _sc_gather(qidx, sub32, _ext_consts(), tabp)
    dense = x[:, _N_EMB:]
    return _tc_project(emb, dense, W[:_EMB_COLS], W[_EMB_COLS:],
                       b.reshape(1, _OUT_DIM))


# speedup vs baseline: 1.0447x; 1.0447x over previous
import functools

import jax
import jax.numpy as jnp
from jax import lax
from jax.experimental import pallas as pl
from jax.experimental.pallas import tpu as pltpu
from jax.experimental.pallas import tpu_sc as plsc

_B = 4096
_N_EMB = 26
_N_DENSE = 13
_VOCAB = 100000
_EMB_DIM = 32
_OUT_DIM = 128
_EMB_COLS = _N_EMB * _EMB_DIM
_PACK = 128 // _EMB_DIM
_VP = _N_EMB * _VOCAB // _PACK

_NC, _NS = 2, 16
_NW = _NC * _NS
_BPW = _B // _NW
_RB = 4
_KC = _RB * _N_EMB
_NCH = _BPW // _RB
_L = 16

_sc_mesh = plsc.VectorSubcoreMesh(core_axis_name="c", subcore_axis_name="s")


@functools.partial(
    pl.kernel,
    out_type=jax.ShapeDtypeStruct((_B, _EMB_COLS), jnp.float32),
    mesh=_sc_mesh,
    scratch_types=[
        pltpu.VMEM((_NCH, _KC), jnp.int32),
        pltpu.VMEM((_KC, _L), jnp.int32),
        pltpu.VMEM((2, _L), jnp.int32),
        pltpu.VMEM((2, _KC, 128), jnp.float32),
        pltpu.VMEM((_RB, _EMB_COLS), jnp.float32),
        pltpu.SemaphoreType.DMA,
        pltpu.SemaphoreType.DMA,
    ],
    compiler_params=pltpu.CompilerParams(needs_layout_passes=False,
                                         use_tc_tiling_on_sc=False),
)
def _sc_gather(qidx_hbm, subs_hbm, io_hbm, tab_hbm, out_hbm,
               qidx_v, subs_v, io_v, slab_v, row_v, sem0, sem1):
    wid = lax.axis_index("s") * _NC + lax.axis_index("c")
    pltpu.sync_copy(qidx_hbm.at[wid], qidx_v)
    pltpu.sync_copy(io_hbm, io_v)
    sems = (sem0, sem1)
    pltpu.async_copy(tab_hbm.at[qidx_v.at[0]], slab_v.at[0], sems[0])

    def half_body(c, p):
        @pl.when(c + 1 < _NCH)
        def _():
            pltpu.async_copy(tab_hbm.at[qidx_v.at[c + 1]], slab_v.at[1 - p],
                             sems[1 - p])
        pltpu.sync_copy(subs_hbm.at[wid, c], subs_v)
        pltpu.make_async_copy(tab_hbm.at[qidx_v.at[c]], slab_v.at[p],
                              sems[p]).wait()
        for k in range(_KC):
            src_row = slab_v.at[p, k]
            dst_row = row_v.at[k // _N_EMB]
            col0 = (k % _N_EMB) * _EMB_DIM
            for h in range(2):
                src_col = subs_v[k, :] + io_v[h, :]
                v = plsc.load_gather(src_row, [src_col])
                plsc.store_scatter(dst_row, [io_v[h, :] + col0], v)
        row0 = wid * _BPW + c * _RB
        pltpu.sync_copy(row_v, out_hbm.at[pl.ds(row0, _RB)])

    def pair_body(g, carry):
        half_body(2 * g, 0)
        half_body(2 * g + 1, 1)
        return carry

    lax.fori_loop(0, _NCH // 2, pair_body, 0)


def _mm_body(emb_ref, dense_ref, w1_ref, w2_ref, b_ref, o_ref):
    acc = jnp.dot(
        emb_ref[...], w1_ref[...],
        preferred_element_type=jnp.float32,
        precision=lax.Precision.HIGHEST,
    )
    acc = acc + jnp.dot(
        dense_ref[...], w2_ref[...],
        preferred_element_type=jnp.float32,
        precision=lax.Precision.HIGHEST,
    )
    o_ref[...] = acc + b_ref[...]


_BM = 512


def _tc_project(emb, dense, w1, w2, b2):
    grid = (_B // _BM,)
    return pl.pallas_call(
        _mm_body,
        grid=grid,
        in_specs=[
            pl.BlockSpec((_BM, _EMB_COLS), lambda i: (i, 0)),
            pl.BlockSpec((_BM, _N_DENSE), lambda i: (i, 0)),
            pl.BlockSpec((_EMB_COLS, _OUT_DIM), lambda i: (0, 0)),
            pl.BlockSpec((_N_DENSE, _OUT_DIM), lambda i: (0, 0)),
            pl.BlockSpec((1, _OUT_DIM), lambda i: (0, 0)),
        ],
        out_specs=pl.BlockSpec((_BM, _OUT_DIM), lambda i: (i, 0)),
        out_shape=jax.ShapeDtypeStruct((_B, _OUT_DIM), jnp.float32),
    )(emb, dense, w1, w2, b2)


def kernel(x, tables, W, b):
    idx = x[:, :_N_EMB].astype(jnp.int32)
    flat = idx + (jnp.arange(_N_EMB, dtype=jnp.int32) * _VOCAB)[None, :]
    qidx = (flat >> 2).reshape(_NW, _NCH, _KC)
    sub32 = ((flat & 3) * _EMB_DIM).reshape(_NW, _NCH, _KC)
    subs = sub32[..., None] + jnp.zeros((_L,), jnp.int32)
    io = jnp.arange(2 * _L, dtype=jnp.int32).reshape(2, _L)
    tabp = tables.reshape(_VP, _PACK * _EMB_DIM)
    emb = _sc_gather(qidx, subs, io, tabp)
    dense = x[:, _N_EMB:]
    return _tc_project(emb, dense, W[:_EMB_COLS], W[_EMB_COLS:],
                       b.reshape(1, _OUT_DIM))
